# baseline (speedup 1.0000x reference)
import jax
import jax.numpy as jnp
from jax.experimental import pallas as pl
from jax.experimental.pallas import tpu as pltpu

_TN = 1000
_H = _TN // 2


def _body(x_hbm, w_ref, b_ref, o_hbm, buf0, buf1, ob0, ob1, in_sem, out_sem):
    bufs = (buf0, buf1)
    outb = (ob0, ob1)
    n, k, d = x_hbm.shape
    o = w_ref.shape[0]
    nsteps = n // _TN
    w = w_ref[...]
    b = b_ref[...]

    def in_copy(i, bslot, h):
        src = x_hbm.at[pl.ds(i * _TN + h * _H, _H)]
        dst = bufs[bslot].at[pl.ds(h * _H, _H)]
        return pltpu.make_async_copy(src, dst, in_sem.at[bslot, h])

    def out_copy(i, bslot):
        return pltpu.make_async_copy(
            outb[bslot], o_hbm.at[pl.ds(i * _TN, _TN)], out_sem.at[bslot])

    for bslot in range(2):
        for h in range(2):
            in_copy(bslot, bslot, h).start()

    pending_out = {}
    for i in range(nsteps):
        bslot = i % 2
        if bslot in pending_out:
            pending_out[bslot].wait()
        for h in range(2):
            in_copy(i, bslot, h).wait()
            x = bufs[bslot][pl.ds(h * _H, _H)].reshape(_H * k, d)
            hm = jax.lax.dot_general(
                x, w,
                dimension_numbers=(((1,), (1,)), ((), ())),
                preferred_element_type=jnp.float32,
            )
            pooled = jnp.max(hm.reshape(_H, k, o), axis=1) + b
            outb[bslot][pl.ds(h * _H, _H)] = jnp.maximum(pooled, 0.0)
        cp = out_copy(i, bslot)
        cp.start()
        pending_out[bslot] = cp
        if i + 2 < nsteps:
            for h in range(2):
                in_copy(i + 2, bslot, h).start()

    for cp in pending_out.values():
        cp.wait()


def kernel(agg_feat, W0, b0):
    n, k, d = agg_feat.shape
    o = W0.shape[0]
    b2 = b0.reshape(1, o)
    return pl.pallas_call(
        _body,
        in_specs=[
            pl.BlockSpec(memory_space=pltpu.MemorySpace.HBM),
            pl.BlockSpec((o, d), lambda: (0, 0)),
            pl.BlockSpec((1, o), lambda: (0, 0)),
        ],
        out_specs=pl.BlockSpec(memory_space=pltpu.MemorySpace.HBM),
        out_shape=jax.ShapeDtypeStruct((n, o), jnp.float32),
        scratch_shapes=(
            [pltpu.VMEM((_TN, k, d), jnp.float32) for _ in range(2)]
            + [pltpu.VMEM((_TN, o), jnp.float32) for _ in range(2)]
            + [pltpu.SemaphoreType.DMA((2, 2)), pltpu.SemaphoreType.DMA((2,))]
        ),
    )(agg_feat, W0, b2)
